# Initial kernel scaffold; baseline (speedup 1.0000x reference)
import jax
import jax.numpy as jnp
from jax.experimental import pallas as pl


def kernel(x, edge_index, batch, params):
    raise NotImplementedError("write your pallas kernel here")


# speedup vs baseline: 2.9877x; 2.9877x over previous
import functools

import jax
import jax.numpy as jnp
from jax import lax
from jax.experimental import pallas as pl
from jax.experimental.pallas import tpu as pltpu
from jax.experimental.pallas import tpu_sc as plsc

N = 10000
E = 160000
DF = 256
H = 512
T = 10
G = 128
BN_EPS = 1e-5

RB = 1000
NRB = N // RB
NCH = 4
CH = H // NCH

NSUB = 16
EPW = E // NSUB
EB = 128
NEB = (EPW + EB - 1) // EB
EPAD = NEB * EB
ACC_ROWS = 10240
PAD_DST = 10100
ZROWS = ACC_ROWS // NSUB

_PREC = lax.Precision.HIGHEST


def _mm(a, b):
    return lax.dot_general(a, b, (((1,), (0,)), ((), ())),
                           preferred_element_type=jnp.float32,
                           precision=_PREC)


def _stats_update(step, z, s_ref):
    @pl.when(step == 0)
    def _():
        s_ref[...] = jnp.zeros_like(s_ref)

    s_ref[0:1, :] += jnp.sum(z, axis=0, keepdims=True)
    s_ref[1:2, :] += jnp.sum(z * z, axis=0, keepdims=True)


def _bn_coefs(s_ref, g_ref, be_ref):
    m = s_ref[0:1, :] * (1.0 / N)
    ex2 = s_ref[1:2, :] * (1.0 / N)
    v = ex2 - m * m
    inv = lax.rsqrt(v + BN_EPS)
    a = g_ref[...] * inv
    c = be_ref[...] - m * a
    return a, c


def _passA0_body(x_ref, w_ref, b_ref, z_ref, s_ref):
    z = _mm(x_ref[...], w_ref[...]) + b_ref[...]
    z_ref[...] = z
    _stats_update(pl.program_id(0), z, s_ref)


def _passA0(x, w1, b1):
    return pl.pallas_call(
        _passA0_body,
        grid=(NRB,),
        in_specs=[
            pl.BlockSpec((RB, DF), lambda r: (r, 0)),
            pl.BlockSpec((DF, H), lambda r: (0, 0)),
            pl.BlockSpec((1, H), lambda r: (0, 0)),
        ],
        out_specs=[
            pl.BlockSpec((RB, H), lambda r: (r, 0)),
            pl.BlockSpec((8, H), lambda r: (0, 0)),
        ],
        out_shape=[
            jax.ShapeDtypeStruct((N, H), jnp.float32),
            jax.ShapeDtypeStruct((8, H), jnp.float32),
        ],
    )(x, w1, b1)


def _passA_body(h_ref, agg_ref, w_ref, b_ref, z_ref, s_ref):
    u = h_ref[...] + jnp.concatenate(
        [agg_ref[0], agg_ref[1], agg_ref[2], agg_ref[3]], axis=1)
    z = _mm(u, w_ref[...]) + b_ref[...]
    z_ref[...] = z
    _stats_update(pl.program_id(0), z, s_ref)


def _passA(h, agg4, w1, b1):
    return pl.pallas_call(
        _passA_body,
        grid=(NRB,),
        in_specs=[
            pl.BlockSpec((RB, H), lambda r: (r, 0)),
            pl.BlockSpec((NCH, RB, CH), lambda r: (0, r, 0)),
            pl.BlockSpec((H, H), lambda r: (0, 0)),
            pl.BlockSpec((1, H), lambda r: (0, 0)),
        ],
        out_specs=[
            pl.BlockSpec((RB, H), lambda r: (r, 0)),
            pl.BlockSpec((8, H), lambda r: (0, 0)),
        ],
        out_shape=[
            jax.ShapeDtypeStruct((N, H), jnp.float32),
            jax.ShapeDtypeStruct((8, H), jnp.float32),
        ],
    )(h, agg4, w1, b1)


def _passB_body(z1_ref, s1_ref, g_ref, be_ref, w_ref, b_ref, z2_ref, s2_ref):
    a, c = _bn_coefs(s1_ref, g_ref, be_ref)
    y = jnp.maximum(z1_ref[...] * a + c, 0.0)
    z2 = _mm(y, w_ref[...]) + b_ref[...]
    z2_ref[...] = z2
    _stats_update(pl.program_id(0), z2, s2_ref)


def _passB(z1, s1, g1, be1, w2, b2):
    return pl.pallas_call(
        _passB_body,
        grid=(NRB,),
        in_specs=[
            pl.BlockSpec((RB, H), lambda r: (r, 0)),
            pl.BlockSpec((8, H), lambda r: (0, 0)),
            pl.BlockSpec((1, H), lambda r: (0, 0)),
            pl.BlockSpec((1, H), lambda r: (0, 0)),
            pl.BlockSpec((H, H), lambda r: (0, 0)),
            pl.BlockSpec((1, H), lambda r: (0, 0)),
        ],
        out_specs=[
            pl.BlockSpec((RB, H), lambda r: (r, 0)),
            pl.BlockSpec((8, H), lambda r: (0, 0)),
        ],
        out_shape=[
            jax.ShapeDtypeStruct((N, H), jnp.float32),
            jax.ShapeDtypeStruct((8, H), jnp.float32),
        ],
    )(z1, s1, g1, be1, w2, b2)


def _passC_body(z2_ref, s2_ref, g_ref, be_ref, b3_ref, h_ref, p_ref):
    a, c = _bn_coefs(s2_ref, g_ref, be_ref)
    y = jnp.maximum(z2_ref[...] * a + c, 0.0)
    h_ref[...] = y
    bb = b3_ref[0]
    maskT = (lax.broadcasted_iota(jnp.int32, (G, RB), 0) == bb
             ).astype(jnp.float32)
    contrib = _mm(maskT, y)

    @pl.when(pl.program_id(0) == 0)
    def _():
        p_ref[...] = jnp.zeros_like(p_ref)

    p_ref[...] += contrib


def _passC(z2, s2, g2, be2, batch3):
    return pl.pallas_call(
        _passC_body,
        grid=(NRB,),
        in_specs=[
            pl.BlockSpec((RB, H), lambda r: (r, 0)),
            pl.BlockSpec((8, H), lambda r: (0, 0)),
            pl.BlockSpec((1, H), lambda r: (0, 0)),
            pl.BlockSpec((1, H), lambda r: (0, 0)),
            pl.BlockSpec((1, 1, RB), lambda r: (r, 0, 0)),
        ],
        out_specs=[
            pl.BlockSpec((RB, H), lambda r: (r, 0)),
            pl.BlockSpec((G, H), lambda r: (0, 0)),
        ],
        out_shape=[
            jax.ShapeDtypeStruct((N, H), jnp.float32),
            jax.ShapeDtypeStruct((G, H), jnp.float32),
        ],
    )(z2, s2, g2, be2, batch3)


def _final_body(p5_ref, lw5_ref, lb5_ref, b3_ref, o_ref):
    acc = jnp.zeros((G, T), jnp.float32)
    for i in range(5):
        acc = acc + _mm(p5_ref[i], lw5_ref[i])
    cnt = jnp.zeros((G, 1), jnp.float32)
    for r in range(NRB):
        bb = b3_ref[r]
        mT = (lax.broadcasted_iota(jnp.int32, (G, RB), 0) == bb
              ).astype(jnp.float32)
        cnt = cnt + jnp.sum(mT, axis=1, keepdims=True)
    acc = acc + cnt * lb5_ref[0]
    acc = acc + (lb5_ref[1] + lb5_ref[2] + lb5_ref[3] + lb5_ref[4])
    o_ref[...] = acc


def _final(p5, lw5, lb5, batch3):
    return pl.pallas_call(
        _final_body,
        in_specs=[
            pl.BlockSpec((5, G, H), lambda: (0, 0, 0)),
            pl.BlockSpec((5, H, T), lambda: (0, 0, 0)),
            pl.BlockSpec((5, 1, T), lambda: (0, 0, 0)),
            pl.BlockSpec((NRB, 1, RB), lambda: (0, 0, 0)),
        ],
        out_specs=pl.BlockSpec((G, T), lambda: (0, 0)),
        out_shape=jax.ShapeDtypeStruct((G, T), jnp.float32),
    )(p5, lw5, lb5, batch3)


def _sc_agg(h2d, sidx4, didx):
    mesh = plsc.VectorSubcoreMesh(core_axis_name="c", subcore_axis_name="s")

    @functools.partial(
        pl.kernel, mesh=mesh,
        out_type=jax.ShapeDtypeStruct((NCH, ACC_ROWS, CH), jnp.float32),
        scratch_types=[
            pltpu.VMEM((NEB, EB), jnp.int32),
            pltpu.VMEM((NEB, EB), jnp.int32),
            pltpu.VMEM((EB, CH), jnp.float32),
            pltpu.VMEM((64, CH), jnp.float32),
            pltpu.VMEM_SHARED((ACC_ROWS, CH), jnp.float32),
        ],
    )
    def k(h_hbm, sidx_hbm, didx_hbm, out_hbm,
          sbuf, dbuf, rbuf, zbuf, acc):
        cid = lax.axis_index("c")
        sid = lax.axis_index("s")

        @pl.loop(0, 64)
        def _(i):
            for kk in range(CH // 16):
                zbuf[i, pl.ds(kk * 16, 16)] = jnp.zeros((16,), jnp.float32)

        pltpu.sync_copy(didx_hbm.at[sid], dbuf)

        for cc in range(2):
            chunk = cid * 2 + cc
            base = sid * ZROWS
            for kk in range(ZROWS // 64):
                pltpu.sync_copy(zbuf,
                                acc.at[pl.ds(base + kk * 64, 64)])
            pltpu.sync_copy(sidx_hbm.at[sid], sbuf)

            @pl.loop(0, NEB)
            def _(j):
                for kk in range(EB // 16):
                    sbuf[j, pl.ds(kk * 16, 16)] = (
                        sbuf[j, pl.ds(kk * 16, 16)] + chunk)

            plsc.subcore_barrier()

            @pl.loop(0, NEB)
            def _(j):
                pltpu.sync_copy(h_hbm.at[sbuf.at[j]], rbuf)
                pltpu.sync_copy(rbuf, acc.at[dbuf.at[j]], add=True)

            plsc.subcore_barrier()
            pltpu.sync_copy(
                acc.at[pl.ds(sid * ZROWS, ZROWS)],
                out_hbm.at[chunk].at[pl.ds(sid * ZROWS, ZROWS)])
            plsc.subcore_barrier()

    return k(h2d, sidx4, didx)


def _mlp0(x, p, batch3):
    z1, s1 = _passA0(x, p["w1"], p["b1"].reshape(1, H))
    z2, s2 = _passB(z1, s1, p["g1"].reshape(1, H), p["be1"].reshape(1, H),
                    p["w2"], p["b2"].reshape(1, H))
    return _passC(z2, s2, p["g2"].reshape(1, H), p["be2"].reshape(1, H),
                  batch3)


def _mlp(h, agg4, p, batch3):
    z1, s1 = _passA(h, agg4, p["w1"], p["b1"].reshape(1, H))
    z2, s2 = _passB(z1, s1, p["g1"].reshape(1, H), p["be1"].reshape(1, H),
                    p["w2"], p["b2"].reshape(1, H))
    return _passC(z2, s2, p["g2"].reshape(1, H), p["be2"].reshape(1, H),
                  batch3)


def kernel(x, edge_index, batch, params):
    src = edge_index[0].astype(jnp.int32)
    dst = edge_index[1].astype(jnp.int32)
    srcp = jnp.pad(src.reshape(NSUB, EPW), ((0, 0), (0, EPAD - EPW)))
    dstp = jnp.pad(dst.reshape(NSUB, EPW), ((0, 0), (0, EPAD - EPW)),
                   constant_values=PAD_DST)
    sidx4 = (srcp * NCH).reshape(NSUB, NEB, EB)
    didx = dstp.reshape(NSUB, NEB, EB)
    batch3 = batch.astype(jnp.int32).reshape(NRB, 1, RB)

    h, pooled0 = _mlp0(x, params["first_h"], batch3)
    pooled = [pooled0]
    for i in range(4):
        agg4 = _sc_agg(h.reshape(N * NCH, CH), sidx4, didx)
        h, p_i = _mlp(h, agg4, params["nns"][i], batch3)
        pooled.append(p_i)

    p5 = jnp.stack(pooled)
    lw5 = jnp.stack(params["lin_w"])
    lb5 = jnp.stack(params["lin_b"]).reshape(5, 1, T)
    return _final(p5, lw5, lb5, batch3)
